# baseline (speedup 1.0000x reference)
import jax
import jax.numpy as jnp
from jax.experimental import pallas as pl
from jax.experimental.pallas import tpu as pltpu

N = 32768
D = 4096
E = 64
TAU = 1.0

TILE = 1024


def _router_kernel(x_ref, w_in_ref, b_ref, sel_ref, aux_ref, probs_ref,
                   logits_ref):
    logits = jax.lax.dot_general(
        x_ref[...], w_in_ref[...], (((1,), (1,)), ((), ())),
        preferred_element_type=jnp.float32,
    )
    logits = logits + b_ref[...]
    inv_tau = 1.0 / max(TAU, 1e-06)
    if inv_tau != 1.0:
        logits = logits * inv_tau

    iota = jax.lax.broadcasted_iota(jnp.int32, logits.shape, 1)
    m1 = jnp.max(logits, axis=1, keepdims=True)
    idx1 = jnp.min(jnp.where(logits == m1, iota, E), axis=1, keepdims=True)
    masked = jnp.where(iota == idx1, -jnp.inf, logits)
    m2 = jnp.max(masked, axis=1, keepdims=True)
    idx2 = jnp.min(jnp.where(masked == m2, iota, E), axis=1, keepdims=True)

    w1 = 1.0 / (1.0 + jnp.exp(m2 - m1))
    aux = jnp.concatenate(
        [idx1.astype(jnp.float32), idx2.astype(jnp.float32), w1, 1.0 - w1],
        axis=1,
    )
    aux_ref[...] = aux.T

    p = jnp.exp(logits - m1)
    probs = p / jnp.sum(p, axis=1, keepdims=True)

    sel = sel_ref[...]
    sel_e = sel[:, :TILE]
    sel_o = sel[:, TILE:]
    for vals, out in ((logits, logits_ref), (probs, probs_ref)):
        out[:, :E] = jax.lax.dot_general(
            sel_e, vals, (((1,), (0,)), ((), ())),
            preferred_element_type=jnp.float32,
        )
        out[:, E:] = jax.lax.dot_general(
            sel_o, vals, (((1,), (0,)), ((), ())),
            preferred_element_type=jnp.float32,
        )


@jax.jit
def kernel(x, W, b):
    b2 = b.reshape(1, E)
    rows = jax.lax.broadcasted_iota(jnp.int32, (TILE // 2, 2 * TILE), 0)
    cols = jax.lax.broadcasted_iota(jnp.int32, (TILE // 2, 2 * TILE), 1)
    sel = jnp.where(
        cols < TILE,
        (cols == 2 * rows).astype(jnp.float32),
        (cols - TILE == 2 * rows + 1).astype(jnp.float32),
    )
    grid = (N // TILE,)
    out_shapes = (
        jax.ShapeDtypeStruct((4, N), jnp.float32),
        jax.ShapeDtypeStruct((N // 2, 2 * E), jnp.float32),
        jax.ShapeDtypeStruct((N // 2, 2 * E), jnp.float32),
    )
    pair_spec = pl.BlockSpec((TILE // 2, 2 * E), lambda i: (i, 0))
    aux_t, probs_pair, logits_pair = pl.pallas_call(
        _router_kernel,
        grid=grid,
        in_specs=[
            pl.BlockSpec((TILE, D), lambda i: (i, 0)),
            pl.BlockSpec((E, D), lambda i: (0, 0)),
            pl.BlockSpec((1, E), lambda i: (0, 0)),
            pl.BlockSpec((TILE // 2, 2 * TILE), lambda i: (0, 0)),
        ],
        out_specs=(
            pl.BlockSpec((4, TILE), lambda i: (0, i)),
            pair_spec,
            pair_spec,
        ),
        out_shape=out_shapes,
        compiler_params=pltpu.CompilerParams(
            dimension_semantics=("arbitrary",),
        ),
    )(x, W, b2, sel)
    top_idx = aux_t[:2].T.astype(jnp.int32)
    top_w = aux_t[2:].T
    probs_full = probs_pair.reshape(N, E)
    logits = logits_pair.reshape(N, E)
    return (top_idx, top_w, probs_full, logits)


# speedup vs baseline: 1.1721x; 1.1721x over previous
import jax
import jax.numpy as jnp
from jax.experimental import pallas as pl
from jax.experimental.pallas import tpu as pltpu

N = 32768
D = 4096
E = 64
TAU = 1.0

TILE = 512


def _router_kernel(x_ref, w_in_ref, b_ref, aux_ref, probs_ref, logits_ref):
    logits = jax.lax.dot_general(
        x_ref[...], w_in_ref[...], (((1,), (1,)), ((), ())),
        preferred_element_type=jnp.float32,
    )
    logits = logits + b_ref[...]
    inv_tau = 1.0 / max(TAU, 1e-06)
    if inv_tau != 1.0:
        logits = logits * inv_tau
    logits_ref[...] = logits

    iota = jax.lax.broadcasted_iota(jnp.int32, logits.shape, 1)
    m1 = jnp.max(logits, axis=1, keepdims=True)
    idx1 = jnp.min(jnp.where(logits == m1, iota, E), axis=1, keepdims=True)
    masked = jnp.where(iota == idx1, -jnp.inf, logits)
    m2 = jnp.max(masked, axis=1, keepdims=True)
    idx2 = jnp.min(jnp.where(masked == m2, iota, E), axis=1, keepdims=True)

    w1 = 1.0 / (1.0 + jnp.exp(m2 - m1))
    aux = jnp.concatenate(
        [idx1.astype(jnp.float32), idx2.astype(jnp.float32), w1, 1.0 - w1],
        axis=1,
    )
    aux_ref[...] = aux.T

    p = jnp.exp(logits - m1)
    probs_ref[...] = p / jnp.sum(p, axis=1, keepdims=True)


@jax.jit
def kernel(x, W, b):
    b2 = b.reshape(1, E)
    grid = (N // TILE,)
    out_shapes = (
        jax.ShapeDtypeStruct((4, N), jnp.float32),
        jax.ShapeDtypeStruct((N, E), jnp.float32),
        jax.ShapeDtypeStruct((N, E), jnp.float32),
    )
    row_specE = pl.BlockSpec((TILE, E), lambda i: (i, 0))
    aux_t, probs_full, logits = pl.pallas_call(
        _router_kernel,
        grid=grid,
        in_specs=[
            pl.BlockSpec((TILE, D), lambda i: (i, 0)),
            pl.BlockSpec((E, D), lambda i: (0, 0)),
            pl.BlockSpec((1, E), lambda i: (0, 0)),
        ],
        out_specs=(
            pl.BlockSpec((4, TILE), lambda i: (0, i)),
            row_specE,
            row_specE,
        ),
        out_shape=out_shapes,
        compiler_params=pltpu.CompilerParams(
            dimension_semantics=("arbitrary",),
        ),
    )(x, W, b2)
    top_idx = aux_t[:2].T.astype(jnp.int32)
    top_w = aux_t[2:].T
    return (top_idx, top_w, probs_full, logits)


# speedup vs baseline: 1.2699x; 1.0834x over previous
import jax
import jax.numpy as jnp
from jax.experimental import pallas as pl
from jax.experimental.pallas import tpu as pltpu

N = 32768
D = 4096
E = 64
TAU = 1.0

TILE = 1024


def _router_kernel(x_ref, w_in_ref, b_ref, aux_ref, probs_ref, logits_ref):
    logits = jax.lax.dot_general(
        x_ref[...], w_in_ref[...], (((1,), (1,)), ((), ())),
        preferred_element_type=jnp.float32,
    )
    logits = logits + b_ref[...]
    inv_tau = 1.0 / max(TAU, 1e-06)
    if inv_tau != 1.0:
        logits = logits * inv_tau
    logits_ref[...] = logits

    iota = jax.lax.broadcasted_iota(jnp.int32, logits.shape, 1)
    m1 = jnp.max(logits, axis=1, keepdims=True)
    idx1 = jnp.min(jnp.where(logits == m1, iota, E), axis=1, keepdims=True)
    masked = jnp.where(iota == idx1, -jnp.inf, logits)
    m2 = jnp.max(masked, axis=1, keepdims=True)
    idx2 = jnp.min(jnp.where(masked == m2, iota, E), axis=1, keepdims=True)

    w1 = 1.0 / (1.0 + jnp.exp(m2 - m1))
    aux = jnp.concatenate(
        [idx1.astype(jnp.float32), idx2.astype(jnp.float32), w1, 1.0 - w1],
        axis=1,
    )
    aux_ref[...] = aux.T

    p = jnp.exp(logits - m1)
    probs_ref[...] = p / jnp.sum(p, axis=1, keepdims=True)


@jax.jit
def kernel(x, W, b):
    b2 = b.reshape(1, E)
    grid = (N // TILE,)
    out_shapes = (
        jax.ShapeDtypeStruct((4, N), jnp.float32),
        jax.ShapeDtypeStruct((N, E), jnp.float32),
        jax.ShapeDtypeStruct((N, E), jnp.float32),
    )
    row_specE = pl.BlockSpec((TILE, E), lambda i: (i, 0))
    aux_t, probs_full, logits = pl.pallas_call(
        _router_kernel,
        grid=grid,
        in_specs=[
            pl.BlockSpec((TILE, D), lambda i: (i, 0)),
            pl.BlockSpec((E, D), lambda i: (0, 0)),
            pl.BlockSpec((1, E), lambda i: (0, 0)),
        ],
        out_specs=(
            pl.BlockSpec((4, TILE), lambda i: (0, i)),
            row_specE,
            row_specE,
        ),
        out_shape=out_shapes,
        compiler_params=pltpu.CompilerParams(
            dimension_semantics=("arbitrary",),
        ),
    )(x, W, b2)
    top_idx = aux_t[:2].T.astype(jnp.int32)
    top_w = aux_t[2:].T
    return (top_idx, top_w, probs_full, logits)


# speedup vs baseline: 1.2722x; 1.0018x over previous
import jax
import jax.numpy as jnp
from jax.experimental import pallas as pl
from jax.experimental.pallas import tpu as pltpu

N = 32768
D = 4096
E = 64
TAU = 1.0

TILE = 1024


def _router_kernel(x_ref, w_in_ref, b_ref, aux_ref, probs_ref, logits_ref):
    logits = jax.lax.dot_general(
        x_ref[...], w_in_ref[...], (((1,), (1,)), ((), ())),
        preferred_element_type=jnp.float32,
    )
    logits = logits + b_ref[...]
    inv_tau = 1.0 / max(TAU, 1e-06)
    if inv_tau != 1.0:
        logits = logits * inv_tau
    logits_ref[...] = logits

    iota = jax.lax.broadcasted_iota(jnp.int32, logits.shape, 1)
    m1 = jnp.max(logits, axis=1, keepdims=True)
    idx1 = jnp.min(jnp.where(logits == m1, iota, E), axis=1, keepdims=True)
    masked = jnp.where(iota == idx1, -jnp.inf, logits)
    m2 = jnp.max(masked, axis=1, keepdims=True)
    idx2 = jnp.min(jnp.where(masked == m2, iota, E), axis=1, keepdims=True)

    w1 = 1.0 / (1.0 + jnp.exp(m2 - m1))
    aux = jnp.concatenate(
        [idx1.astype(jnp.float32), idx2.astype(jnp.float32), w1, 1.0 - w1],
        axis=1,
    )
    aux_ref[...] = aux.T

    p = jnp.exp(logits - m1)
    probs_ref[...] = p / jnp.sum(p, axis=1, keepdims=True)


@jax.jit
def kernel(x, W, b):
    b2 = b.reshape(1, E)
    grid = (N // TILE,)
    out_shapes = (
        jax.ShapeDtypeStruct((4, N), jnp.float32),
        jax.ShapeDtypeStruct((N, E), jnp.float32),
        jax.ShapeDtypeStruct((N, E), jnp.float32),
    )
    row_specE = pl.BlockSpec((TILE, E), lambda i: (i, 0))
    aux_t, probs_full, logits = pl.pallas_call(
        _router_kernel,
        grid=grid,
        in_specs=[
            pl.BlockSpec((TILE, D), lambda i: (i, 0)),
            pl.BlockSpec((E, D), lambda i: (0, 0)),
            pl.BlockSpec((1, E), lambda i: (0, 0)),
        ],
        out_specs=(
            pl.BlockSpec((4, TILE), lambda i: (0, i)),
            row_specE,
            row_specE,
        ),
        out_shape=out_shapes,
        compiler_params=pltpu.CompilerParams(
            dimension_semantics=("parallel",),
        ),
    )(x, W, b2)
    top_idx = aux_t[:2].T.astype(jnp.int32)
    top_w = aux_t[2:].T
    return (top_idx, top_w, probs_full, logits)
